# Initial kernel scaffold; baseline (speedup 1.0000x reference)
import functools

import jax
import jax.numpy as jnp
from jax import lax
from jax.experimental import pallas as pl
from jax.experimental.pallas import tpu as pltpu
from jax.experimental.pallas import tpu_sc as plsc

N = 10000
E = 320000
IN_CH = 128
HID_CH = 128
OUT_CH = 64

NC = 2
NS = 16
NW = NC * NS
CHUNK = 128
CHUNKS = 80
EPW = CHUNK * CHUNKS
E_PAD = EPW * NW
ROWS_PER_TILE = 632
N_TAB = ROWS_PER_TILE * NS
TRASH = N
DEG_W = 16

_MESH = plsc.VectorSubcoreMesh(core_axis_name="c", subcore_axis_name="s")


@functools.partial(
    pl.kernel,
    out_type=jax.ShapeDtypeStruct((NC, N_TAB, DEG_W), jnp.float32),
    mesh=_MESH,
    scratch_types=[
        pltpu.VMEM((CHUNKS, CHUNK), jnp.int32),
        pltpu.VMEM((CHUNK, DEG_W), jnp.float32),
        pltpu.VMEM_SHARED((N_TAB, DEG_W), jnp.float32),
    ],
)
def _deg_kernel(dst_hbm, ones_hbm, zeros_hbm, out_hbm, dst_v, ones_v, table):
    c = lax.axis_index("c")
    s = lax.axis_index("s")
    w = s * NC + c
    pltpu.sync_copy(dst_hbm.at[w], dst_v)
    pltpu.sync_copy(ones_hbm, ones_v)
    row0 = s * ROWS_PER_TILE
    pltpu.sync_copy(zeros_hbm, table.at[pl.ds(row0, ROWS_PER_TILE)])
    plsc.subcore_barrier()

    def body(j, carry):
        pltpu.sync_copy(ones_v, table.at[dst_v.at[j]], add=True)
        return carry

    lax.fori_loop(0, CHUNKS, body, 0)
    plsc.subcore_barrier()
    pltpu.sync_copy(
        table.at[pl.ds(row0, ROWS_PER_TILE)],
        out_hbm.at[c, pl.ds(row0, ROWS_PER_TILE)],
    )


def _make_agg_kernel(d):
    @functools.partial(
        pl.kernel,
        out_type=jax.ShapeDtypeStruct((NC, N_TAB, d), jnp.float32),
        mesh=_MESH,
        scratch_types=[
            pltpu.VMEM((CHUNKS, CHUNK), jnp.int32),
            pltpu.VMEM((CHUNKS, CHUNK), jnp.int32),
            pltpu.VMEM((CHUNK, d), jnp.float32),
            pltpu.VMEM((CHUNK, d), jnp.float32),
            pltpu.VMEM_SHARED((N_TAB, d), jnp.float32),
            pltpu.SemaphoreType.DMA,
            pltpu.SemaphoreType.DMA,
        ],
    )
    def _agg(u_hbm, src_hbm, dst_hbm, zeros_hbm, out_hbm,
             src_v, dst_v, buf0, buf1, table, sem0, sem1):
        c = lax.axis_index("c")
        s = lax.axis_index("s")
        w = s * NC + c
        pltpu.sync_copy(src_hbm.at[w], src_v)
        pltpu.sync_copy(dst_hbm.at[w], dst_v)
        row0 = s * ROWS_PER_TILE
        pltpu.sync_copy(zeros_hbm, table.at[pl.ds(row0, ROWS_PER_TILE)])
        plsc.subcore_barrier()

        pltpu.async_copy(u_hbm.at[src_v.at[0]], buf0, sem0)

        def body(jj, carry):
            def step(j, buf, sem, nbuf, nsem):
                pltpu.async_copy(u_hbm.at[src_v.at[j + 1]], nbuf, nsem)
                pltpu.make_async_copy(u_hbm.at[src_v.at[j]], buf, sem).wait()
                pltpu.sync_copy(buf, table.at[dst_v.at[j]], add=True)

            j = jj * 2
            step(j, buf0, sem0, buf1, sem1)
            step(j + 1, buf1, sem1, buf0, sem0)
            return carry

        lax.fori_loop(0, CHUNKS // 2 - 1, body, 0)
        j = CHUNKS - 2
        pltpu.async_copy(u_hbm.at[src_v.at[j + 1]], buf1, sem1)
        pltpu.make_async_copy(u_hbm.at[src_v.at[j]], buf0, sem0).wait()
        pltpu.sync_copy(buf0, table.at[dst_v.at[j]], add=True)
        pltpu.make_async_copy(u_hbm.at[src_v.at[j + 1]], buf1, sem1).wait()
        pltpu.sync_copy(buf1, table.at[dst_v.at[j + 1]], add=True)

        plsc.subcore_barrier()
        pltpu.sync_copy(
            table.at[pl.ds(row0, ROWS_PER_TILE)],
            out_hbm.at[c, pl.ds(row0, ROWS_PER_TILE)],
        )

    return _agg


_agg128 = _make_agg_kernel(HID_CH)
_agg64 = _make_agg_kernel(OUT_CH)


_R = 2000


def _dinv_block(dega_ref, degb_ref):
    deg = dega_ref[:, 0:1] + degb_ref[:, 0:1] + 1.0
    return lax.rsqrt(deg)


def _lin1_body(x_ref, w_ref, dega_ref, degb_ref, u_ref):
    dinv = _dinv_block(dega_ref, degb_ref)
    h = jnp.dot(x_ref[...], w_ref[...], preferred_element_type=jnp.float32)
    u_ref[...] = h * dinv


def _mid_body(agg0_ref, agg1_ref, u1_ref, dega_ref, degb_ref, b1_ref, w2_ref,
              u2_ref):
    dinv = _dinv_block(dega_ref, degb_ref)
    t = (agg0_ref[...] + agg1_ref[...] + u1_ref[...]) * dinv + b1_ref[...]
    h1 = jnp.maximum(t, 0.0)
    u2_ref[...] = jnp.dot(h1, w2_ref[...],
                          preferred_element_type=jnp.float32) * dinv


def _fin_body(agg0_ref, agg1_ref, u2_ref, dega_ref, degb_ref, b2_ref, o_ref):
    dinv = _dinv_block(dega_ref, degb_ref)
    z = (agg0_ref[...] + agg1_ref[...] + u2_ref[...]) * dinv + b2_ref[...]
    m = jnp.max(z, axis=1, keepdims=True)
    lse = jnp.log(jnp.sum(jnp.exp(z - m), axis=1, keepdims=True)) + m
    o_ref[...] = z - lse


def _row_spec(width):
    return pl.BlockSpec((_R, width), lambda i: (i, 0))


def _full_spec(shape):
    return pl.BlockSpec(shape, lambda i: tuple(0 for _ in shape))


def _lin1(x, w1, dega, degb):
    return pl.pallas_call(
        _lin1_body,
        grid=(N // _R,),
        in_specs=[
            _row_spec(IN_CH),
            _full_spec((IN_CH, HID_CH)),
            _row_spec(DEG_W),
            _row_spec(DEG_W),
        ],
        out_specs=_row_spec(HID_CH),
        out_shape=jax.ShapeDtypeStruct((N, HID_CH), jnp.float32),
    )(x, w1, dega, degb)


def _mid(agg0, agg1, u1, dega, degb, b1, w2):
    return pl.pallas_call(
        _mid_body,
        grid=(N // _R,),
        in_specs=[
            _row_spec(HID_CH),
            _row_spec(HID_CH),
            _row_spec(HID_CH),
            _row_spec(DEG_W),
            _row_spec(DEG_W),
            _full_spec((1, HID_CH)),
            _full_spec((HID_CH, OUT_CH)),
        ],
        out_specs=_row_spec(OUT_CH),
        out_shape=jax.ShapeDtypeStruct((N, OUT_CH), jnp.float32),
    )(agg0, agg1, u1, dega, degb, b1, w2)


def _fin(agg0, agg1, u2, dega, degb, b2):
    return pl.pallas_call(
        _fin_body,
        grid=(N // _R,),
        in_specs=[
            _row_spec(OUT_CH),
            _row_spec(OUT_CH),
            _row_spec(OUT_CH),
            _row_spec(DEG_W),
            _row_spec(DEG_W),
            _full_spec((1, OUT_CH)),
        ],
        out_specs=_row_spec(OUT_CH),
        out_shape=jax.ShapeDtypeStruct((N, OUT_CH), jnp.float32),
    )(agg0, agg1, u2, dega, degb, b2)


@jax.jit
def kernel(x, edge_index, W1, b1, W2, b2):
    src = edge_index[0].astype(jnp.int32)
    dst = edge_index[1].astype(jnp.int32)
    pad = E_PAD - E
    src_p = jnp.concatenate(
        [src, jnp.zeros((pad,), jnp.int32)]).reshape(NW, CHUNKS, CHUNK)
    dst_p = jnp.concatenate(
        [dst, jnp.full((pad,), TRASH, jnp.int32)]).reshape(NW, CHUNKS, CHUNK)

    ones_deg = jnp.ones((CHUNK, DEG_W), jnp.float32)
    zeros_deg = jnp.zeros((ROWS_PER_TILE, DEG_W), jnp.float32)
    zeros_h = jnp.zeros((ROWS_PER_TILE, HID_CH), jnp.float32)
    zeros_o = jnp.zeros((ROWS_PER_TILE, OUT_CH), jnp.float32)

    deg_parts = _deg_kernel(dst_p, ones_deg, zeros_deg)
    dega, degb = deg_parts[0], deg_parts[1]

    u1 = _lin1(x, W1, dega, degb)
    agg1 = _agg128(u1, src_p, dst_p, zeros_h)
    u2 = _mid(agg1[0, :N], agg1[1, :N], u1, dega, degb,
              b1.reshape(1, HID_CH), W2)
    agg2 = _agg64(u2, src_p, dst_p, zeros_o)
    return _fin(agg2[0, :N], agg2[1, :N], u2, dega, degb,
                b2.reshape(1, OUT_CH))


# speedup vs baseline: 8.7451x; 8.7451x over previous
import functools

import jax
import jax.numpy as jnp
from jax import lax
from jax.experimental import pallas as pl
from jax.experimental.pallas import tpu as pltpu
from jax.experimental.pallas import tpu_sc as plsc

N = 10000
E = 320000
IN_CH = 128
HID_CH = 128
OUT_CH = 64

NC = 2
NS = 16
NW = NC * NS
CHUNK = 128
CHUNKS = 80
GRP = 8
NGRP = CHUNKS // GRP
EPW = CHUNK * CHUNKS
E_PAD = EPW * NW
ROWS_PER_TILE = 632
N_TAB = ROWS_PER_TILE * NS
TRASH = N
DEG_W = 16

_MESH = plsc.VectorSubcoreMesh(core_axis_name="c", subcore_axis_name="s")


@functools.partial(
    pl.kernel,
    out_type=jax.ShapeDtypeStruct((NC, N_TAB, DEG_W), jnp.float32),
    mesh=_MESH,
    scratch_types=[
        pltpu.VMEM((CHUNKS, CHUNK), jnp.int32),
        pltpu.VMEM((CHUNK, DEG_W), jnp.float32),
        pltpu.VMEM_SHARED((N_TAB, DEG_W), jnp.float32),
    ],
)
def _deg_kernel(dst_hbm, ones_hbm, zeros_hbm, out_hbm, dst_v, ones_v, table):
    c = lax.axis_index("c")
    s = lax.axis_index("s")
    w = s * NC + c
    pltpu.sync_copy(dst_hbm.at[w], dst_v)
    pltpu.sync_copy(ones_hbm, ones_v)
    row0 = s * ROWS_PER_TILE
    pltpu.sync_copy(zeros_hbm, table.at[pl.ds(row0, ROWS_PER_TILE)])
    plsc.subcore_barrier()

    def body(j, carry):
        pltpu.sync_copy(ones_v, table.at[dst_v.at[j]], add=True)
        return carry

    lax.fori_loop(0, CHUNKS, body, 0)
    plsc.subcore_barrier()
    pltpu.sync_copy(
        table.at[pl.ds(row0, ROWS_PER_TILE)],
        out_hbm.at[c, pl.ds(row0, ROWS_PER_TILE)],
    )


def _make_agg_kernel(d):
    @functools.partial(
        pl.kernel,
        out_type=jax.ShapeDtypeStruct((NC, N_TAB, d), jnp.float32),
        mesh=_MESH,
        scratch_types=[
            pltpu.VMEM((GRP, CHUNK), jnp.int32),
            pltpu.VMEM((GRP, CHUNK), jnp.int32),
            pltpu.VMEM((CHUNK, d), jnp.float32),
            pltpu.VMEM((CHUNK, d), jnp.float32),
            pltpu.VMEM_SHARED((N_TAB, d), jnp.float32),
            pltpu.SemaphoreType.DMA,
            pltpu.SemaphoreType.DMA,
        ],
    )
    def _agg(u_hbm, src_hbm, dst_hbm, zeros_hbm, out_hbm,
             src_v, dst_v, buf0, buf1, table, sem0, sem1):
        c = lax.axis_index("c")
        s = lax.axis_index("s")
        w = s * NC + c
        row0 = s * ROWS_PER_TILE
        pltpu.sync_copy(zeros_hbm, table.at[pl.ds(row0, ROWS_PER_TILE)])
        plsc.subcore_barrier()

        bufs = (buf0, buf1)
        sems = (sem0, sem1)

        def group(g, carry):
            pltpu.sync_copy(src_hbm.at[w, pl.ds(g * GRP, GRP)], src_v)
            pltpu.sync_copy(dst_hbm.at[w, pl.ds(g * GRP, GRP)], dst_v)
            pltpu.async_copy(u_hbm.at[src_v.at[0]], buf0, sem0)
            for j in range(GRP):
                if j + 1 < GRP:
                    pltpu.async_copy(u_hbm.at[src_v.at[j + 1]],
                                     bufs[(j + 1) % 2], sems[(j + 1) % 2])
                pltpu.make_async_copy(u_hbm.at[src_v.at[j]],
                                      bufs[j % 2], sems[j % 2]).wait()
                pltpu.sync_copy(bufs[j % 2], table.at[dst_v.at[j]], add=True)
            return carry

        lax.fori_loop(0, NGRP, group, 0)

        plsc.subcore_barrier()
        pltpu.sync_copy(
            table.at[pl.ds(row0, ROWS_PER_TILE)],
            out_hbm.at[c, pl.ds(row0, ROWS_PER_TILE)],
        )

    return _agg


_agg128 = _make_agg_kernel(HID_CH)


_R = 2000


def _dinv_block(dega_ref, degb_ref):
    deg = dega_ref[:, 0:1] + degb_ref[:, 0:1] + 1.0
    return lax.rsqrt(deg)


def _lin1_body(x_ref, w_ref, dega_ref, degb_ref, u_ref):
    dinv = _dinv_block(dega_ref, degb_ref)
    h = jnp.dot(x_ref[...], w_ref[...], preferred_element_type=jnp.float32)
    u_ref[...] = h * dinv


def _mid_body(agg0_ref, agg1_ref, u1_ref, dega_ref, degb_ref, b1_ref, w2_ref,
              u2_ref):
    dinv = _dinv_block(dega_ref, degb_ref)
    t = (agg0_ref[...] + agg1_ref[...] + u1_ref[...]) * dinv + b1_ref[...]
    h1 = jnp.maximum(t, 0.0)
    u2 = jnp.dot(h1, w2_ref[...], preferred_element_type=jnp.float32) * dinv
    u2_ref[...] = jnp.pad(u2, ((0, 0), (0, HID_CH - OUT_CH)))


def _fin_body(agg0_ref, agg1_ref, u2_ref, dega_ref, degb_ref, b2_ref, o_ref):
    dinv = _dinv_block(dega_ref, degb_ref)
    z = (agg0_ref[...] + agg1_ref[...] + u2_ref[...]) * dinv + b2_ref[...]
    m = jnp.max(z, axis=1, keepdims=True)
    lse = jnp.log(jnp.sum(jnp.exp(z - m), axis=1, keepdims=True)) + m
    o_ref[...] = z - lse


def _row_spec(width):
    return pl.BlockSpec((_R, width), lambda i: (i, 0))


def _full_spec(shape):
    return pl.BlockSpec(shape, lambda i: tuple(0 for _ in shape))


def _lin1(x, w1, dega, degb):
    return pl.pallas_call(
        _lin1_body,
        grid=(N // _R,),
        in_specs=[
            _row_spec(IN_CH),
            _full_spec((IN_CH, HID_CH)),
            _row_spec(DEG_W),
            _row_spec(DEG_W),
        ],
        out_specs=_row_spec(HID_CH),
        out_shape=jax.ShapeDtypeStruct((N, HID_CH), jnp.float32),
    )(x, w1, dega, degb)


def _mid(agg0, agg1, u1, dega, degb, b1, w2):
    return pl.pallas_call(
        _mid_body,
        grid=(N // _R,),
        in_specs=[
            _row_spec(HID_CH),
            _row_spec(HID_CH),
            _row_spec(HID_CH),
            _row_spec(DEG_W),
            _row_spec(DEG_W),
            _full_spec((1, HID_CH)),
            _full_spec((HID_CH, OUT_CH)),
        ],
        out_specs=_row_spec(HID_CH),
        out_shape=jax.ShapeDtypeStruct((N, HID_CH), jnp.float32),
    )(agg0, agg1, u1, dega, degb, b1, w2)


def _fin(agg0, agg1, u2, dega, degb, b2):
    return pl.pallas_call(
        _fin_body,
        grid=(N // _R,),
        in_specs=[
            _row_spec(OUT_CH),
            _row_spec(OUT_CH),
            _row_spec(OUT_CH),
            _row_spec(DEG_W),
            _row_spec(DEG_W),
            _full_spec((1, OUT_CH)),
        ],
        out_specs=_row_spec(OUT_CH),
        out_shape=jax.ShapeDtypeStruct((N, OUT_CH), jnp.float32),
    )(agg0, agg1, u2, dega, degb, b2)


@jax.jit
def kernel(x, edge_index, W1, b1, W2, b2):
    src = edge_index[0].astype(jnp.int32)
    dst = edge_index[1].astype(jnp.int32)
    pad = E_PAD - E
    src_p = jnp.concatenate(
        [src, jnp.zeros((pad,), jnp.int32)]).reshape(NW, CHUNKS, CHUNK)
    dst_p = jnp.concatenate(
        [dst, jnp.full((pad,), TRASH, jnp.int32)]).reshape(NW, CHUNKS, CHUNK)

    ones_deg = jnp.ones((CHUNK, DEG_W), jnp.float32)
    zeros_deg = jnp.zeros((ROWS_PER_TILE, DEG_W), jnp.float32)
    zeros_h = jnp.zeros((ROWS_PER_TILE, HID_CH), jnp.float32)

    deg_parts = _deg_kernel(dst_p, ones_deg, zeros_deg)
    dega, degb = deg_parts[0], deg_parts[1]

    u1 = _lin1(x, W1, dega, degb)
    agg1 = _agg128(u1, src_p, dst_p, zeros_h)
    u2 = _mid(agg1[0, :N], agg1[1, :N], u1, dega, degb,
              b1.reshape(1, HID_CH), W2)
    agg2 = _agg128(u2, src_p, dst_p, zeros_h)
    return _fin(agg2[0, :N, :OUT_CH], agg2[1, :N, :OUT_CH], u2[:, :OUT_CH],
                dega, degb, b2.reshape(1, OUT_CH))


# speedup vs baseline: 26.2593x; 3.0027x over previous
import functools

import jax
import jax.numpy as jnp
from jax import lax
from jax.experimental import pallas as pl
from jax.experimental.pallas import tpu as pltpu
from jax.experimental.pallas import tpu_sc as plsc

N = 10000
E = 320000
IN_CH = 128
HID_CH = 128
OUT_CH = 64

NC = 2
NS = 16
NW = NC * NS
CHUNK = 128
CHUNKS = 80
GRP = 8
NGRP = CHUNKS // GRP
EPW = CHUNK * CHUNKS
E_PAD = EPW * NW
ROWS_PER_TILE = 632
N_TAB = ROWS_PER_TILE * NS
TRASH = N
DEG_W = 16

_MESH = plsc.VectorSubcoreMesh(core_axis_name="c", subcore_axis_name="s")


@functools.partial(
    pl.kernel,
    out_type=jax.ShapeDtypeStruct((NC, N_TAB, DEG_W), jnp.float32),
    mesh=_MESH,
    scratch_types=[
        pltpu.VMEM((CHUNKS, CHUNK), jnp.int32),
        pltpu.VMEM((CHUNK, DEG_W), jnp.float32),
        pltpu.VMEM_SHARED((N_TAB, DEG_W), jnp.float32),
    ],
)
def _deg_kernel(dst_hbm, ones_hbm, zeros_hbm, out_hbm, dst_v, ones_v, table):
    c = lax.axis_index("c")
    s = lax.axis_index("s")
    w = s * NC + c
    pltpu.sync_copy(dst_hbm.at[w], dst_v)
    pltpu.sync_copy(ones_hbm, ones_v)
    row0 = s * ROWS_PER_TILE
    pltpu.sync_copy(zeros_hbm, table.at[pl.ds(row0, ROWS_PER_TILE)])
    plsc.subcore_barrier()

    def body(j, carry):
        pltpu.sync_copy(ones_v, table.at[dst_v.at[j]], add=True)
        return carry

    lax.fori_loop(0, CHUNKS, body, 0)
    plsc.subcore_barrier()
    pltpu.sync_copy(
        table.at[pl.ds(row0, ROWS_PER_TILE)],
        out_hbm.at[c, pl.ds(row0, ROWS_PER_TILE)],
    )


def _make_agg_kernel(d):
    @functools.partial(
        pl.kernel,
        out_type=jax.ShapeDtypeStruct((NC, N_TAB, d), jnp.float32),
        mesh=_MESH,
        scratch_types=[
            pltpu.VMEM((GRP, CHUNK), jnp.int32),
            pltpu.VMEM((GRP, CHUNK), jnp.int32),
            pltpu.VMEM((CHUNK, d), jnp.float32),
            pltpu.VMEM((CHUNK, d), jnp.float32),
            pltpu.VMEM_SHARED((N_TAB, d), jnp.float32),
            pltpu.SemaphoreType.DMA,
            pltpu.SemaphoreType.DMA,
        ],
    )
    def _agg(u_hbm, src_hbm, dst_hbm, zeros_hbm, out_hbm,
             src_v, dst_v, buf0, buf1, table, sem0, sem1):
        c = lax.axis_index("c")
        s = lax.axis_index("s")
        w = s * NC + c
        row0 = s * ROWS_PER_TILE
        pltpu.sync_copy(zeros_hbm, table.at[pl.ds(row0, ROWS_PER_TILE)])
        plsc.subcore_barrier()

        bufs = (buf0, buf1)
        sems = (sem0, sem1)

        def group(g, carry):
            pltpu.sync_copy(src_hbm.at[w, pl.ds(g * GRP, GRP)], src_v)
            pltpu.sync_copy(dst_hbm.at[w, pl.ds(g * GRP, GRP)], dst_v)
            pltpu.async_copy(u_hbm.at[src_v.at[0]], buf0, sem0)
            for j in range(GRP):
                if j + 1 < GRP:
                    pltpu.async_copy(u_hbm.at[src_v.at[j + 1]],
                                     bufs[(j + 1) % 2], sems[(j + 1) % 2])
                pltpu.make_async_copy(u_hbm.at[src_v.at[j]],
                                      bufs[j % 2], sems[j % 2]).wait()
                pltpu.sync_copy(bufs[j % 2], table.at[dst_v.at[j]], add=True)
            return carry

        lax.fori_loop(0, NGRP, group, 0)

        plsc.subcore_barrier()
        pltpu.sync_copy(
            table.at[pl.ds(row0, ROWS_PER_TILE)],
            out_hbm.at[c, pl.ds(row0, ROWS_PER_TILE)],
        )

    return _agg


_agg128 = _make_agg_kernel(HID_CH)


_R = 2000


def _dinv_block(dega_ref, degb_ref):
    deg = dega_ref[:, 0:1] + degb_ref[:, 0:1] + 1.0
    return lax.rsqrt(deg)


def _lin1_body(x_ref, w_ref, dega_ref, degb_ref, u_ref):
    dinv = _dinv_block(dega_ref, degb_ref)
    h = jnp.dot(x_ref[...], w_ref[...], preferred_element_type=jnp.float32)
    u_ref[...] = h * dinv


def _mid_body(agg0_ref, agg1_ref, u1_ref, dega_ref, degb_ref, b1_ref, w2_ref,
              u2_ref):
    dinv = _dinv_block(dega_ref, degb_ref)
    t = (agg0_ref[...] + agg1_ref[...] + u1_ref[...]) * dinv + b1_ref[...]
    h1 = jnp.maximum(t, 0.0)
    u2 = jnp.dot(h1, w2_ref[...], preferred_element_type=jnp.float32) * dinv
    u2_ref[...] = jnp.pad(u2, ((0, 0), (0, HID_CH - OUT_CH)))


def _fin_body(agg0_ref, agg1_ref, u2_ref, dega_ref, degb_ref, b2_ref, o_ref):
    dinv = _dinv_block(dega_ref, degb_ref)
    z = (agg0_ref[...] + agg1_ref[...] + u2_ref[...]) * dinv + b2_ref[...]
    m = jnp.max(z, axis=1, keepdims=True)
    lse = jnp.log(jnp.sum(jnp.exp(z - m), axis=1, keepdims=True)) + m
    o_ref[...] = z - lse


def _row_spec(width):
    return pl.BlockSpec((_R, width), lambda i: (i, 0))


def _full_spec(shape):
    return pl.BlockSpec(shape, lambda i: tuple(0 for _ in shape))


def _lin1(x, w1, dega, degb):
    return pl.pallas_call(
        _lin1_body,
        grid=(N // _R,),
        in_specs=[
            _row_spec(IN_CH),
            _full_spec((IN_CH, HID_CH)),
            _row_spec(DEG_W),
            _row_spec(DEG_W),
        ],
        out_specs=_row_spec(HID_CH),
        out_shape=jax.ShapeDtypeStruct((N, HID_CH), jnp.float32),
    )(x, w1, dega, degb)


def _mid(agg0, agg1, u1, dega, degb, b1, w2):
    return pl.pallas_call(
        _mid_body,
        grid=(N // _R,),
        in_specs=[
            _row_spec(HID_CH),
            _row_spec(HID_CH),
            _row_spec(HID_CH),
            _row_spec(DEG_W),
            _row_spec(DEG_W),
            _full_spec((1, HID_CH)),
            _full_spec((HID_CH, OUT_CH)),
        ],
        out_specs=_row_spec(HID_CH),
        out_shape=jax.ShapeDtypeStruct((N, HID_CH), jnp.float32),
    )(agg0, agg1, u1, dega, degb, b1, w2)


def _fin(agg0, agg1, u2, dega, degb, b2):
    return pl.pallas_call(
        _fin_body,
        grid=(N // _R,),
        in_specs=[
            _row_spec(OUT_CH),
            _row_spec(OUT_CH),
            _row_spec(OUT_CH),
            _row_spec(DEG_W),
            _row_spec(DEG_W),
            _full_spec((1, OUT_CH)),
        ],
        out_specs=_row_spec(OUT_CH),
        out_shape=jax.ShapeDtypeStruct((N, OUT_CH), jnp.float32),
    )(agg0, agg1, u2, dega, degb, b2)


@jax.jit
def kernel(x, edge_index, W1, b1, W2, b2):
    src = edge_index[0].astype(jnp.int32)
    dst = edge_index[1].astype(jnp.int32)
    pad = E_PAD - E
    pad_i = jnp.arange(pad, dtype=jnp.int32)
    src_p = jnp.concatenate(
        [src, pad_i % N]).reshape(NW, CHUNKS, CHUNK)
    dst_p = jnp.concatenate(
        [dst, TRASH + pad_i % (N_TAB - N)]).reshape(NW, CHUNKS, CHUNK)

    ones_deg = jnp.ones((CHUNK, DEG_W), jnp.float32)
    zeros_deg = jnp.zeros((ROWS_PER_TILE, DEG_W), jnp.float32)
    zeros_h = jnp.zeros((ROWS_PER_TILE, HID_CH), jnp.float32)

    deg_parts = _deg_kernel(dst_p, ones_deg, zeros_deg)
    dega, degb = deg_parts[0], deg_parts[1]

    u1 = _lin1(x, W1, dega, degb)
    agg1 = _agg128(u1, src_p, dst_p, zeros_h)
    u2 = _mid(agg1[0, :N], agg1[1, :N], u1, dega, degb,
              b1.reshape(1, HID_CH), W2)
    agg2 = _agg128(u2, src_p, dst_p, zeros_h)
    return _fin(agg2[0, :N, :OUT_CH], agg2[1, :N, :OUT_CH], u2[:, :OUT_CH],
                dega, degb, b2.reshape(1, OUT_CH))
